# baseline (speedup 1.0000x reference)
import functools

import jax
import jax.numpy as jnp
from jax import lax
from jax.experimental import pallas as pl
from jax.experimental.pallas import tpu as pltpu
from jax.experimental.pallas import tpu_sc as plsc

N_NODES = 10000
N_EDGES = 320000
D_IN = 128
D_HID = 32
D_OUT = 128

NC = 2
NS = 16
NW = NC * NS
CH = 80
CHUNKS_PER_WORKER = N_EDGES // (NW * CH)
ACC_ROWS = 10112
ROWS_PER_SUB = ACC_ROWS // NS
HALF_A = 320
HALF_B = ROWS_PER_SUB - HALF_A
CNT_W = 16
LANES = 16

ROW_BLOCK = 2000

_SC_PARAMS = pltpu.CompilerParams(use_tc_tiling_on_sc=False)


def _l1_body(table, ei_hbm, zeros_s, zeros_c, ones_hbm,
             out_sums, out_cnt,
             src_v, dst_v, rows_v, sem_g, ones_v, sem_o, acc, acc_cnt):
    c = lax.axis_index("c")
    s = lax.axis_index("s")
    wid = s * NC + c

    row0 = s * ROWS_PER_SUB
    pltpu.sync_copy(zeros_s.at[pl.ds(row0, ROWS_PER_SUB)],
                    acc.at[pl.ds(row0, ROWS_PER_SUB)])
    pltpu.sync_copy(zeros_c.at[pl.ds(row0, ROWS_PER_SUB)],
                    acc_cnt.at[pl.ds(row0, ROWS_PER_SUB)])
    pltpu.sync_copy(ones_hbm, ones_v)
    pltpu.sync_copy(ei_hbm.at[0, wid], src_v)
    pltpu.sync_copy(ei_hbm.at[1, wid], dst_v)
    plsc.subcore_barrier()

    pltpu.async_copy(table.at[src_v.at[0]], rows_v.at[0], sem_g)
    pltpu.async_copy(table.at[src_v.at[1]], rows_v.at[1], sem_g)

    def body(i, carry):
        p = lax.rem(i, 4)

        @pl.when(i + 2 < CHUNKS_PER_WORKER)
        def _prefetch():
            pltpu.async_copy(table.at[src_v.at[i + 2]],
                             rows_v.at[lax.rem(i + 2, 4)], sem_g)

        pltpu.make_async_copy(table.at[src_v.at[i]], rows_v.at[p],
                              sem_g).wait()
        pltpu.async_copy(ones_v, acc_cnt.at[dst_v.at[i]], sem_o, add=True)
        pltpu.sync_copy(rows_v.at[p], acc.at[dst_v.at[i]], add=True)
        return carry

    lax.fori_loop(0, CHUNKS_PER_WORKER, body, 0)

    def drain(i, carry):
        pltpu.make_async_copy(ones_v, acc_cnt.at[dst_v.at[0]], sem_o).wait()
        return carry
    lax.fori_loop(0, CHUNKS_PER_WORKER, drain, 0)

    plsc.subcore_barrier()

    pltpu.sync_copy(acc.at[pl.ds(row0, ROWS_PER_SUB)],
                    out_sums.at[c, pl.ds(row0, ROWS_PER_SUB)])
    pltpu.sync_copy(acc_cnt.at[pl.ds(row0, ROWS_PER_SUB)],
                    out_cnt.at[c, pl.ds(row0, ROWS_PER_SUB)])


_seg_sum_counts = pl.kernel(
    _l1_body,
    out_type=[
        jax.ShapeDtypeStruct((NC, ACC_ROWS, D_HID), jnp.float32),
        jax.ShapeDtypeStruct((NC, ACC_ROWS, CNT_W), jnp.float32),
    ],
    mesh=plsc.VectorSubcoreMesh(core_axis_name="c", subcore_axis_name="s"),
    scratch_types=[
        pltpu.VMEM((CHUNKS_PER_WORKER, CH), jnp.int32),
        pltpu.VMEM((CHUNKS_PER_WORKER, CH), jnp.int32),
        pltpu.VMEM((4, CH, D_HID), jnp.float32),
        pltpu.SemaphoreType.DMA,
        pltpu.VMEM((CH, CNT_W), jnp.float32),
        pltpu.SemaphoreType.DMA,
        pltpu.VMEM_SHARED((ACC_ROWS, D_HID), jnp.float32),
        pltpu.VMEM_SHARED((ACC_ROWS, CNT_W), jnp.float32),
    ],
    compiler_params=_SC_PARAMS,
)


def _l2_body(sums1, cnt1, r1_hbm, b1_hbm, ei_hbm, zeros_s,
             out_sums, h_tabs, h_out, inv_out,
             src_v, dst_v, rows_v, sem_g, sem_p,
             s0_v, s1_v, c0_v, c1_v, r_v, b_v, acc):
    c = lax.axis_index("c")
    s = lax.axis_index("s")
    wid = s * NC + c
    h_tab = h_tabs.at[c]

    row0 = s * ROWS_PER_SUB
    pltpu.sync_copy(zeros_s.at[pl.ds(row0, ROWS_PER_SUB)],
                    acc.at[pl.ds(row0, ROWS_PER_SUB)])
    pltpu.sync_copy(ei_hbm.at[0, wid], src_v)
    pltpu.sync_copy(ei_hbm.at[1, wid], dst_v)
    pltpu.sync_copy(b1_hbm, b_v)

    for k, hn in ((0, HALF_A), (1, HALF_B)):
        r0k = row0 + k * HALF_A
        pltpu.async_copy(sums1.at[0, pl.ds(r0k, hn)], s0_v.at[pl.ds(0, hn)],
                         sem_p)
        pltpu.async_copy(sums1.at[1, pl.ds(r0k, hn)], s1_v.at[pl.ds(0, hn)],
                         sem_p)
        pltpu.async_copy(cnt1.at[0, pl.ds(r0k, hn)], c0_v.at[pl.ds(0, hn)],
                         sem_p)
        pltpu.async_copy(cnt1.at[1, pl.ds(r0k, hn)], c1_v.at[pl.ds(0, hn)],
                         sem_p)
        pltpu.async_copy(r1_hbm.at[pl.ds(r0k, hn)], r_v.at[pl.ds(0, hn)],
                         sem_p)
        pltpu.make_async_copy(sums1.at[0, pl.ds(r0k, hn)],
                              s0_v.at[pl.ds(0, hn)], sem_p).wait()
        pltpu.make_async_copy(sums1.at[0, pl.ds(r0k, hn)],
                              s0_v.at[pl.ds(0, hn)], sem_p).wait()
        pltpu.make_async_copy(cnt1.at[0, pl.ds(r0k, hn)],
                              c0_v.at[pl.ds(0, hn)], sem_p).wait()
        pltpu.make_async_copy(cnt1.at[0, pl.ds(r0k, hn)],
                              c0_v.at[pl.ds(0, hn)], sem_p).wait()
        pltpu.make_async_copy(r1_hbm.at[pl.ds(r0k, hn)],
                              r_v.at[pl.ds(0, hn)], sem_p).wait()

        def compute(i, carry):
            cv = c0_v[i, :] + c1_v[i, :]
            inv = 1.0 / jnp.maximum(cv, 1.0)
            c0_v[i, :] = inv
            for j in (0, LANES):
                val = ((s0_v[i, pl.ds(j, LANES)] + s1_v[i, pl.ds(j, LANES)])
                       * inv
                       + b_v[pl.ds(j, LANES)]
                       + r_v[i, pl.ds(j, LANES)])
                r_v[i, pl.ds(j, LANES)] = jnp.maximum(val, 0.0)
            return carry

        lax.fori_loop(0, hn, compute, 0)

        pltpu.sync_copy(r_v.at[pl.ds(0, hn)], h_tab.at[pl.ds(r0k, hn)])

        @pl.when(c == 0)
        def _publish():
            pltpu.sync_copy(r_v.at[pl.ds(0, hn)], h_out.at[pl.ds(r0k, hn)])
            pltpu.sync_copy(c0_v.at[pl.ds(0, hn)], inv_out.at[pl.ds(r0k, hn)])

    plsc.subcore_barrier()

    pltpu.async_copy(h_tab.at[src_v.at[0]], rows_v.at[0], sem_g)
    pltpu.async_copy(h_tab.at[src_v.at[1]], rows_v.at[1], sem_g)

    def body(i, carry):
        p = lax.rem(i, 4)

        @pl.when(i + 2 < CHUNKS_PER_WORKER)
        def _prefetch():
            pltpu.async_copy(h_tab.at[src_v.at[i + 2]],
                             rows_v.at[lax.rem(i + 2, 4)], sem_g)

        pltpu.make_async_copy(h_tab.at[src_v.at[i]], rows_v.at[p],
                              sem_g).wait()
        pltpu.sync_copy(rows_v.at[p], acc.at[dst_v.at[i]], add=True)
        return carry

    lax.fori_loop(0, CHUNKS_PER_WORKER, body, 0)
    plsc.subcore_barrier()

    pltpu.sync_copy(acc.at[pl.ds(row0, ROWS_PER_SUB)],
                    out_sums.at[c, pl.ds(row0, ROWS_PER_SUB)])


_layer2 = pl.kernel(
    _l2_body,
    out_type=[
        jax.ShapeDtypeStruct((NC, ACC_ROWS, D_HID), jnp.float32),
        jax.ShapeDtypeStruct((NC, ACC_ROWS, D_HID), jnp.float32),
        jax.ShapeDtypeStruct((ACC_ROWS, D_HID), jnp.float32),
        jax.ShapeDtypeStruct((ACC_ROWS, CNT_W), jnp.float32),
    ],
    mesh=plsc.VectorSubcoreMesh(core_axis_name="c", subcore_axis_name="s"),
    scratch_types=[
        pltpu.VMEM((CHUNKS_PER_WORKER, CH), jnp.int32),
        pltpu.VMEM((CHUNKS_PER_WORKER, CH), jnp.int32),
        pltpu.VMEM((4, CH, D_HID), jnp.float32),
        pltpu.SemaphoreType.DMA,
        pltpu.SemaphoreType.DMA,
        pltpu.VMEM((HALF_A, D_HID), jnp.float32),
        pltpu.VMEM((HALF_A, D_HID), jnp.float32),
        pltpu.VMEM((HALF_A, CNT_W), jnp.float32),
        pltpu.VMEM((HALF_A, CNT_W), jnp.float32),
        pltpu.VMEM((HALF_A, D_HID), jnp.float32),
        pltpu.VMEM((D_HID,), jnp.float32),
        pltpu.VMEM_SHARED((ACC_ROWS, D_HID), jnp.float32),
    ],
    compiler_params=_SC_PARAMS,
)


def _proj_body(x_ref, wl_ref, wr_ref, p_ref, r_ref):
    x = x_ref[...]
    p_ref[...] = jnp.dot(x, wl_ref[...], preferred_element_type=jnp.float32)
    r_ref[...] = jnp.dot(x, wr_ref[...], preferred_element_type=jnp.float32)


_PROJ_BLOCK = 1264

_proj = pl.pallas_call(
    _proj_body,
    grid=(ACC_ROWS // _PROJ_BLOCK,),
    in_specs=[
        pl.BlockSpec((_PROJ_BLOCK, D_IN), lambda i: (i, 0)),
        pl.BlockSpec((D_IN, D_HID), lambda i: (0, 0)),
        pl.BlockSpec((D_IN, D_HID), lambda i: (0, 0)),
    ],
    out_specs=[
        pl.BlockSpec((_PROJ_BLOCK, D_HID), lambda i: (i, 0)),
        pl.BlockSpec((_PROJ_BLOCK, D_HID), lambda i: (i, 0)),
    ],
    out_shape=[
        jax.ShapeDtypeStruct((ACC_ROWS, D_HID), jnp.float32),
        jax.ShapeDtypeStruct((ACC_ROWS, D_HID), jnp.float32),
    ],
)


def _out_body(sums_ref, inv_ref, h_ref, wl_ref, b_ref, wr_ref, o_ref):
    agg = (sums_ref[0] + sums_ref[1]) * inv_ref[:, 0:1]
    o = (jnp.dot(agg, wl_ref[...], preferred_element_type=jnp.float32)
         + b_ref[...]
         + jnp.dot(h_ref[...], wr_ref[...], preferred_element_type=jnp.float32))
    m = jnp.max(o, axis=1, keepdims=True)
    e = jnp.exp(o - m)
    o_ref[...] = o - m - jnp.log(jnp.sum(e, axis=1, keepdims=True))


_out_final = pl.pallas_call(
    _out_body,
    grid=(N_NODES // ROW_BLOCK,),
    in_specs=[
        pl.BlockSpec((NC, ROW_BLOCK, D_HID), lambda i: (0, i, 0)),
        pl.BlockSpec((ROW_BLOCK, CNT_W), lambda i: (i, 0)),
        pl.BlockSpec((ROW_BLOCK, D_HID), lambda i: (i, 0)),
        pl.BlockSpec((D_HID, D_OUT), lambda i: (0, 0)),
        pl.BlockSpec((1, D_OUT), lambda i: (0, 0)),
        pl.BlockSpec((D_HID, D_OUT), lambda i: (0, 0)),
    ],
    out_specs=pl.BlockSpec((ROW_BLOCK, D_OUT), lambda i: (i, 0)),
    out_shape=jax.ShapeDtypeStruct((N_NODES, D_OUT), jnp.float32),
)


def kernel(x, edge_index, W1_l, b1, W1_r, W2_l, b2, W2_r):
    ei = edge_index.astype(jnp.int32).reshape(2, NW, CHUNKS_PER_WORKER, CH)

    zeros_s = jnp.zeros((ACC_ROWS, D_HID), jnp.float32)
    zeros_c = jnp.zeros((ACC_ROWS, CNT_W), jnp.float32)
    ones_b = jnp.ones((CH, CNT_W), jnp.float32)

    p1, r1 = _proj(x, W1_l, W1_r)
    sums1, cnt = _seg_sum_counts(p1, ei, zeros_s, zeros_c, ones_b)
    sums2, _, h, inv = _layer2(sums1, cnt, r1, b1, ei, zeros_s)
    return _out_final(sums2, inv, h, W2_l, b2.reshape(1, D_OUT), W2_r)


# speedup vs baseline: 1.0914x; 1.0914x over previous
import functools

import jax
import jax.numpy as jnp
from jax import lax
from jax.experimental import pallas as pl
from jax.experimental.pallas import tpu as pltpu
from jax.experimental.pallas import tpu_sc as plsc

N_NODES = 10000
N_EDGES = 320000
D_IN = 128
D_HID = 32
D_OUT = 128

NC = 2
NS = 16
NW = NC * NS
CH = 80
CHUNKS_PER_WORKER = N_EDGES // (NW * CH)
ACC_ROWS = 10112
ROWS_PER_SUB = ACC_ROWS // NS
HALF_A = 320
HALF_B = ROWS_PER_SUB - HALF_A
CNT_W = 16
LANES = 16

ROW_BLOCK = 2000

_SC_PARAMS = pltpu.CompilerParams(use_tc_tiling_on_sc=False)


def _l1_body(table, ei_hbm, zeros_s, zeros_c, ones_hbm,
             out_sums, out_cnt,
             src_v, dst_v, rows_v, sem_g, ones_v, sem_o, acc, acc_cnt):
    c = lax.axis_index("c")
    s = lax.axis_index("s")
    wid = s * NC + c

    row0 = s * ROWS_PER_SUB
    pltpu.sync_copy(zeros_s.at[pl.ds(row0, ROWS_PER_SUB)],
                    acc.at[pl.ds(row0, ROWS_PER_SUB)])
    pltpu.sync_copy(zeros_c.at[pl.ds(row0, ROWS_PER_SUB)],
                    acc_cnt.at[pl.ds(row0, ROWS_PER_SUB)])
    pltpu.sync_copy(ones_hbm, ones_v)
    pltpu.sync_copy(ei_hbm.at[0, wid], src_v)
    pltpu.sync_copy(ei_hbm.at[1, wid], dst_v)
    plsc.subcore_barrier()

    pltpu.async_copy(table.at[src_v.at[0]], rows_v.at[0], sem_g)
    pltpu.async_copy(table.at[src_v.at[1]], rows_v.at[1], sem_g)
    pltpu.async_copy(table.at[src_v.at[2]], rows_v.at[2], sem_g)

    def body(i, carry):
        p = lax.rem(i, 4)

        @pl.when(i + 3 < CHUNKS_PER_WORKER)
        def _prefetch():
            pltpu.async_copy(table.at[src_v.at[i + 3]],
                             rows_v.at[lax.rem(i + 3, 4)], sem_g)

        pltpu.make_async_copy(table.at[src_v.at[i]], rows_v.at[p],
                              sem_g).wait()
        pltpu.async_copy(ones_v, acc_cnt.at[dst_v.at[i]], sem_o, add=True)
        pltpu.sync_copy(rows_v.at[p], acc.at[dst_v.at[i]], add=True)
        return carry

    lax.fori_loop(0, CHUNKS_PER_WORKER, body, 0)

    def drain(i, carry):
        pltpu.make_async_copy(ones_v, acc_cnt.at[dst_v.at[0]], sem_o).wait()
        return carry
    lax.fori_loop(0, CHUNKS_PER_WORKER, drain, 0)

    plsc.subcore_barrier()

    pltpu.sync_copy(acc.at[pl.ds(row0, ROWS_PER_SUB)],
                    out_sums.at[c, pl.ds(row0, ROWS_PER_SUB)])
    pltpu.sync_copy(acc_cnt.at[pl.ds(row0, ROWS_PER_SUB)],
                    out_cnt.at[c, pl.ds(row0, ROWS_PER_SUB)])


_seg_sum_counts = pl.kernel(
    _l1_body,
    out_type=[
        jax.ShapeDtypeStruct((NC, ACC_ROWS, D_HID), jnp.float32),
        jax.ShapeDtypeStruct((NC, ACC_ROWS, CNT_W), jnp.float32),
    ],
    mesh=plsc.VectorSubcoreMesh(core_axis_name="c", subcore_axis_name="s"),
    scratch_types=[
        pltpu.VMEM((CHUNKS_PER_WORKER, CH), jnp.int32),
        pltpu.VMEM((CHUNKS_PER_WORKER, CH), jnp.int32),
        pltpu.VMEM((4, CH, D_HID), jnp.float32),
        pltpu.SemaphoreType.DMA,
        pltpu.VMEM((CH, CNT_W), jnp.float32),
        pltpu.SemaphoreType.DMA,
        pltpu.VMEM_SHARED((ACC_ROWS, D_HID), jnp.float32),
        pltpu.VMEM_SHARED((ACC_ROWS, CNT_W), jnp.float32),
    ],
    compiler_params=_SC_PARAMS,
)


def _l2_body(sums1, cnt1, r1_hbm, b1_hbm, ei_hbm, zeros_s,
             out_sums, h_out, inv_out,
             src_v, dst_v, rows_v, sem_g, sem_p,
             s0_v, s1_v, c0_v, c1_v, r_v, b_v, h_tab, acc):
    c = lax.axis_index("c")
    s = lax.axis_index("s")
    wid = s * NC + c

    row0 = s * ROWS_PER_SUB
    pltpu.sync_copy(zeros_s.at[pl.ds(row0, ROWS_PER_SUB)],
                    acc.at[pl.ds(row0, ROWS_PER_SUB)])
    pltpu.sync_copy(ei_hbm.at[0, wid], src_v)
    pltpu.sync_copy(ei_hbm.at[1, wid], dst_v)
    pltpu.sync_copy(b1_hbm, b_v)

    for k, hn in ((0, HALF_A), (1, HALF_B)):
        r0k = row0 + k * HALF_A
        pltpu.async_copy(sums1.at[0, pl.ds(r0k, hn)], s0_v.at[pl.ds(0, hn)],
                         sem_p)
        pltpu.async_copy(sums1.at[1, pl.ds(r0k, hn)], s1_v.at[pl.ds(0, hn)],
                         sem_p)
        pltpu.async_copy(cnt1.at[0, pl.ds(r0k, hn)], c0_v.at[pl.ds(0, hn)],
                         sem_p)
        pltpu.async_copy(cnt1.at[1, pl.ds(r0k, hn)], c1_v.at[pl.ds(0, hn)],
                         sem_p)
        pltpu.async_copy(r1_hbm.at[pl.ds(r0k, hn)], r_v.at[pl.ds(0, hn)],
                         sem_p)
        pltpu.make_async_copy(sums1.at[0, pl.ds(r0k, hn)],
                              s0_v.at[pl.ds(0, hn)], sem_p).wait()
        pltpu.make_async_copy(sums1.at[0, pl.ds(r0k, hn)],
                              s0_v.at[pl.ds(0, hn)], sem_p).wait()
        pltpu.make_async_copy(cnt1.at[0, pl.ds(r0k, hn)],
                              c0_v.at[pl.ds(0, hn)], sem_p).wait()
        pltpu.make_async_copy(cnt1.at[0, pl.ds(r0k, hn)],
                              c0_v.at[pl.ds(0, hn)], sem_p).wait()
        pltpu.make_async_copy(r1_hbm.at[pl.ds(r0k, hn)],
                              r_v.at[pl.ds(0, hn)], sem_p).wait()

        def compute(i, carry):
            cv = c0_v[i, :] + c1_v[i, :]
            inv = 1.0 / jnp.maximum(cv, 1.0)
            c0_v[i, :] = inv
            for j in (0, LANES):
                val = ((s0_v[i, pl.ds(j, LANES)] + s1_v[i, pl.ds(j, LANES)])
                       * inv
                       + b_v[pl.ds(j, LANES)]
                       + r_v[i, pl.ds(j, LANES)])
                r_v[i, pl.ds(j, LANES)] = jnp.maximum(val, 0.0)
            return carry

        lax.fori_loop(0, hn, compute, 0)

        pltpu.sync_copy(r_v.at[pl.ds(0, hn)], h_tab.at[pl.ds(r0k, hn)])

        @pl.when(c == 0)
        def _publish():
            pltpu.sync_copy(r_v.at[pl.ds(0, hn)], h_out.at[pl.ds(r0k, hn)])
            pltpu.sync_copy(c0_v.at[pl.ds(0, hn)], inv_out.at[pl.ds(r0k, hn)])

    plsc.subcore_barrier()

    pltpu.async_copy(h_tab.at[src_v.at[0]], rows_v.at[0], sem_g)
    pltpu.async_copy(h_tab.at[src_v.at[1]], rows_v.at[1], sem_g)
    pltpu.async_copy(h_tab.at[src_v.at[2]], rows_v.at[2], sem_g)

    def body(i, carry):
        p = lax.rem(i, 4)

        @pl.when(i + 3 < CHUNKS_PER_WORKER)
        def _prefetch():
            pltpu.async_copy(h_tab.at[src_v.at[i + 3]],
                             rows_v.at[lax.rem(i + 3, 4)], sem_g)

        pltpu.make_async_copy(h_tab.at[src_v.at[i]], rows_v.at[p],
                              sem_g).wait()
        pltpu.sync_copy(rows_v.at[p], acc.at[dst_v.at[i]], add=True)
        return carry

    lax.fori_loop(0, CHUNKS_PER_WORKER, body, 0)
    plsc.subcore_barrier()

    pltpu.sync_copy(acc.at[pl.ds(row0, ROWS_PER_SUB)],
                    out_sums.at[c, pl.ds(row0, ROWS_PER_SUB)])


_layer2 = pl.kernel(
    _l2_body,
    out_type=[
        jax.ShapeDtypeStruct((NC, ACC_ROWS, D_HID), jnp.float32),
        jax.ShapeDtypeStruct((ACC_ROWS, D_HID), jnp.float32),
        jax.ShapeDtypeStruct((ACC_ROWS, CNT_W), jnp.float32),
    ],
    mesh=plsc.VectorSubcoreMesh(core_axis_name="c", subcore_axis_name="s"),
    scratch_types=[
        pltpu.VMEM((CHUNKS_PER_WORKER, CH), jnp.int32),
        pltpu.VMEM((CHUNKS_PER_WORKER, CH), jnp.int32),
        pltpu.VMEM((4, CH, D_HID), jnp.float32),
        pltpu.SemaphoreType.DMA,
        pltpu.SemaphoreType.DMA,
        pltpu.VMEM((HALF_A, D_HID), jnp.float32),
        pltpu.VMEM((HALF_A, D_HID), jnp.float32),
        pltpu.VMEM((HALF_A, CNT_W), jnp.float32),
        pltpu.VMEM((HALF_A, CNT_W), jnp.float32),
        pltpu.VMEM((HALF_A, D_HID), jnp.float32),
        pltpu.VMEM((D_HID,), jnp.float32),
        pltpu.VMEM_SHARED((ACC_ROWS, D_HID), jnp.float32),
        pltpu.VMEM_SHARED((ACC_ROWS, D_HID), jnp.float32),
    ],
    compiler_params=_SC_PARAMS,
)


def _proj_body(x_ref, wl_ref, wr_ref, p_ref, r_ref):
    x = x_ref[...]
    p_ref[...] = jnp.dot(x, wl_ref[...], preferred_element_type=jnp.float32)
    r_ref[...] = jnp.dot(x, wr_ref[...], preferred_element_type=jnp.float32)


_PROJ_BLOCK = 1264

_proj = pl.pallas_call(
    _proj_body,
    grid=(ACC_ROWS // _PROJ_BLOCK,),
    in_specs=[
        pl.BlockSpec((_PROJ_BLOCK, D_IN), lambda i: (i, 0)),
        pl.BlockSpec((D_IN, D_HID), lambda i: (0, 0)),
        pl.BlockSpec((D_IN, D_HID), lambda i: (0, 0)),
    ],
    out_specs=[
        pl.BlockSpec((_PROJ_BLOCK, D_HID), lambda i: (i, 0)),
        pl.BlockSpec((_PROJ_BLOCK, D_HID), lambda i: (i, 0)),
    ],
    out_shape=[
        jax.ShapeDtypeStruct((ACC_ROWS, D_HID), jnp.float32),
        jax.ShapeDtypeStruct((ACC_ROWS, D_HID), jnp.float32),
    ],
)


def _out_body(sums_ref, inv_ref, h_ref, wl_ref, b_ref, wr_ref, o_ref):
    agg = (sums_ref[0] + sums_ref[1]) * inv_ref[:, 0:1]
    o = (jnp.dot(agg, wl_ref[...], preferred_element_type=jnp.float32)
         + b_ref[...]
         + jnp.dot(h_ref[...], wr_ref[...], preferred_element_type=jnp.float32))
    m = jnp.max(o, axis=1, keepdims=True)
    e = jnp.exp(o - m)
    o_ref[...] = o - m - jnp.log(jnp.sum(e, axis=1, keepdims=True))


_out_final = pl.pallas_call(
    _out_body,
    grid=(N_NODES // ROW_BLOCK,),
    in_specs=[
        pl.BlockSpec((NC, ROW_BLOCK, D_HID), lambda i: (0, i, 0)),
        pl.BlockSpec((ROW_BLOCK, CNT_W), lambda i: (i, 0)),
        pl.BlockSpec((ROW_BLOCK, D_HID), lambda i: (i, 0)),
        pl.BlockSpec((D_HID, D_OUT), lambda i: (0, 0)),
        pl.BlockSpec((1, D_OUT), lambda i: (0, 0)),
        pl.BlockSpec((D_HID, D_OUT), lambda i: (0, 0)),
    ],
    out_specs=pl.BlockSpec((ROW_BLOCK, D_OUT), lambda i: (i, 0)),
    out_shape=jax.ShapeDtypeStruct((N_NODES, D_OUT), jnp.float32),
)


def kernel(x, edge_index, W1_l, b1, W1_r, W2_l, b2, W2_r):
    ei = edge_index.astype(jnp.int32).reshape(2, NW, CHUNKS_PER_WORKER, CH)

    zeros_s = jnp.zeros((ACC_ROWS, D_HID), jnp.float32)
    zeros_c = jnp.zeros((ACC_ROWS, CNT_W), jnp.float32)
    ones_b = jnp.ones((CH, CNT_W), jnp.float32)

    p1, r1 = _proj(x, W1_l, W1_r)
    sums1, cnt = _seg_sum_counts(p1, ei, zeros_s, zeros_c, ones_b)
    sums2, h, inv = _layer2(sums1, cnt, r1, b1, ei, zeros_s)
    return _out_final(sums2, inv, h, W2_l, b2.reshape(1, D_OUT), W2_r)


# speedup vs baseline: 1.1213x; 1.0274x over previous
import functools

import jax
import jax.numpy as jnp
from jax import lax
from jax.experimental import pallas as pl
from jax.experimental.pallas import tpu as pltpu
from jax.experimental.pallas import tpu_sc as plsc

N_NODES = 10000
N_EDGES = 320000
D_IN = 128
D_HID = 32
D_OUT = 128

NC = 2
NS = 16
NW = NC * NS
CH = 80
CHUNKS_PER_WORKER = N_EDGES // (NW * CH)
ACC_ROWS = 10112
ROWS_PER_SUB = ACC_ROWS // NS
HALF_A = 320
HALF_B = ROWS_PER_SUB - HALF_A
CNT_W = 16
LANES = 16

ROW_BLOCK = 2000

_SC_PARAMS = pltpu.CompilerParams(use_tc_tiling_on_sc=False)


def _l1_body(table, ei_hbm, zeros_s, zeros_c, ones_hbm,
             out_sums, out_cnt,
             src_v, dst_v, rows_v, sem_g, ones_v, sem_o, acc, acc_cnt):
    c = lax.axis_index("c")
    s = lax.axis_index("s")
    wid = s * NC + c

    row0 = s * ROWS_PER_SUB
    pltpu.sync_copy(zeros_s.at[pl.ds(row0, ROWS_PER_SUB)],
                    acc.at[pl.ds(row0, ROWS_PER_SUB)])
    pltpu.sync_copy(zeros_c.at[pl.ds(row0, ROWS_PER_SUB)],
                    acc_cnt.at[pl.ds(row0, ROWS_PER_SUB)])
    pltpu.sync_copy(ones_hbm, ones_v)
    pltpu.sync_copy(ei_hbm.at[0, wid], src_v)
    pltpu.sync_copy(ei_hbm.at[1, wid], dst_v)
    plsc.subcore_barrier()

    for _w in range(5):
        pltpu.async_copy(table.at[src_v.at[_w]], rows_v.at[_w], sem_g)

    def body(i, carry):
        p = lax.rem(i, 6)

        @pl.when(i + 5 < CHUNKS_PER_WORKER)
        def _prefetch():
            pltpu.async_copy(table.at[src_v.at[i + 5]],
                             rows_v.at[lax.rem(i + 5, 6)], sem_g)

        pltpu.make_async_copy(table.at[src_v.at[i]], rows_v.at[p],
                              sem_g).wait()
        pltpu.async_copy(ones_v, acc_cnt.at[dst_v.at[i]], sem_o, add=True)
        pltpu.sync_copy(rows_v.at[p], acc.at[dst_v.at[i]], add=True)
        return carry

    lax.fori_loop(0, CHUNKS_PER_WORKER, body, 0)

    def drain(i, carry):
        pltpu.make_async_copy(ones_v, acc_cnt.at[dst_v.at[0]], sem_o).wait()
        return carry
    lax.fori_loop(0, CHUNKS_PER_WORKER, drain, 0)

    plsc.subcore_barrier()

    pltpu.sync_copy(acc.at[pl.ds(row0, ROWS_PER_SUB)],
                    out_sums.at[c, pl.ds(row0, ROWS_PER_SUB)])
    pltpu.sync_copy(acc_cnt.at[pl.ds(row0, ROWS_PER_SUB)],
                    out_cnt.at[c, pl.ds(row0, ROWS_PER_SUB)])


_seg_sum_counts = pl.kernel(
    _l1_body,
    out_type=[
        jax.ShapeDtypeStruct((NC, ACC_ROWS, D_HID), jnp.float32),
        jax.ShapeDtypeStruct((NC, ACC_ROWS, CNT_W), jnp.float32),
    ],
    mesh=plsc.VectorSubcoreMesh(core_axis_name="c", subcore_axis_name="s"),
    scratch_types=[
        pltpu.VMEM((CHUNKS_PER_WORKER, CH), jnp.int32),
        pltpu.VMEM((CHUNKS_PER_WORKER, CH), jnp.int32),
        pltpu.VMEM((6, CH, D_HID), jnp.float32),
        pltpu.SemaphoreType.DMA,
        pltpu.VMEM((CH, CNT_W), jnp.float32),
        pltpu.SemaphoreType.DMA,
        pltpu.VMEM_SHARED((ACC_ROWS, D_HID), jnp.float32),
        pltpu.VMEM_SHARED((ACC_ROWS, CNT_W), jnp.float32),
    ],
    compiler_params=_SC_PARAMS,
)


def _l2_body(sums1, cnt1, r1_hbm, b1_hbm, ei_hbm, zeros_s,
             out_sums, h_out, inv_out,
             src_v, dst_v, rows_v, sem_g, sem_p,
             s0_v, s1_v, c0_v, c1_v, r_v, b_v, h_tab, acc):
    c = lax.axis_index("c")
    s = lax.axis_index("s")
    wid = s * NC + c

    row0 = s * ROWS_PER_SUB
    pltpu.sync_copy(zeros_s.at[pl.ds(row0, ROWS_PER_SUB)],
                    acc.at[pl.ds(row0, ROWS_PER_SUB)])
    pltpu.sync_copy(ei_hbm.at[0, wid], src_v)
    pltpu.sync_copy(ei_hbm.at[1, wid], dst_v)
    pltpu.sync_copy(b1_hbm, b_v)

    for k, hn in ((0, HALF_A), (1, HALF_B)):
        r0k = row0 + k * HALF_A
        pltpu.async_copy(sums1.at[0, pl.ds(r0k, hn)], s0_v.at[pl.ds(0, hn)],
                         sem_p)
        pltpu.async_copy(sums1.at[1, pl.ds(r0k, hn)], s1_v.at[pl.ds(0, hn)],
                         sem_p)
        pltpu.async_copy(cnt1.at[0, pl.ds(r0k, hn)], c0_v.at[pl.ds(0, hn)],
                         sem_p)
        pltpu.async_copy(cnt1.at[1, pl.ds(r0k, hn)], c1_v.at[pl.ds(0, hn)],
                         sem_p)
        pltpu.async_copy(r1_hbm.at[pl.ds(r0k, hn)], r_v.at[pl.ds(0, hn)],
                         sem_p)
        pltpu.make_async_copy(sums1.at[0, pl.ds(r0k, hn)],
                              s0_v.at[pl.ds(0, hn)], sem_p).wait()
        pltpu.make_async_copy(sums1.at[0, pl.ds(r0k, hn)],
                              s0_v.at[pl.ds(0, hn)], sem_p).wait()
        pltpu.make_async_copy(cnt1.at[0, pl.ds(r0k, hn)],
                              c0_v.at[pl.ds(0, hn)], sem_p).wait()
        pltpu.make_async_copy(cnt1.at[0, pl.ds(r0k, hn)],
                              c0_v.at[pl.ds(0, hn)], sem_p).wait()
        pltpu.make_async_copy(r1_hbm.at[pl.ds(r0k, hn)],
                              r_v.at[pl.ds(0, hn)], sem_p).wait()

        def compute(i, carry):
            cv = c0_v[i, :] + c1_v[i, :]
            inv = 1.0 / jnp.maximum(cv, 1.0)
            c0_v[i, :] = inv
            for j in (0, LANES):
                val = ((s0_v[i, pl.ds(j, LANES)] + s1_v[i, pl.ds(j, LANES)])
                       * inv
                       + b_v[pl.ds(j, LANES)]
                       + r_v[i, pl.ds(j, LANES)])
                r_v[i, pl.ds(j, LANES)] = jnp.maximum(val, 0.0)
            return carry

        lax.fori_loop(0, hn, compute, 0)

        pltpu.sync_copy(r_v.at[pl.ds(0, hn)], h_tab.at[pl.ds(r0k, hn)])

        @pl.when(c == 0)
        def _publish():
            pltpu.sync_copy(r_v.at[pl.ds(0, hn)], h_out.at[pl.ds(r0k, hn)])
            pltpu.sync_copy(c0_v.at[pl.ds(0, hn)], inv_out.at[pl.ds(r0k, hn)])

    plsc.subcore_barrier()

    for _w in range(5):
        pltpu.async_copy(h_tab.at[src_v.at[_w]], rows_v.at[_w], sem_g)

    def body(i, carry):
        p = lax.rem(i, 6)

        @pl.when(i + 5 < CHUNKS_PER_WORKER)
        def _prefetch():
            pltpu.async_copy(h_tab.at[src_v.at[i + 5]],
                             rows_v.at[lax.rem(i + 5, 6)], sem_g)

        pltpu.make_async_copy(h_tab.at[src_v.at[i]], rows_v.at[p],
                              sem_g).wait()
        pltpu.sync_copy(rows_v.at[p], acc.at[dst_v.at[i]], add=True)
        return carry

    lax.fori_loop(0, CHUNKS_PER_WORKER, body, 0)
    plsc.subcore_barrier()

    pltpu.sync_copy(acc.at[pl.ds(row0, ROWS_PER_SUB)],
                    out_sums.at[c, pl.ds(row0, ROWS_PER_SUB)])


_layer2 = pl.kernel(
    _l2_body,
    out_type=[
        jax.ShapeDtypeStruct((NC, ACC_ROWS, D_HID), jnp.float32),
        jax.ShapeDtypeStruct((ACC_ROWS, D_HID), jnp.float32),
        jax.ShapeDtypeStruct((ACC_ROWS, CNT_W), jnp.float32),
    ],
    mesh=plsc.VectorSubcoreMesh(core_axis_name="c", subcore_axis_name="s"),
    scratch_types=[
        pltpu.VMEM((CHUNKS_PER_WORKER, CH), jnp.int32),
        pltpu.VMEM((CHUNKS_PER_WORKER, CH), jnp.int32),
        pltpu.VMEM((6, CH, D_HID), jnp.float32),
        pltpu.SemaphoreType.DMA,
        pltpu.SemaphoreType.DMA,
        pltpu.VMEM((HALF_A, D_HID), jnp.float32),
        pltpu.VMEM((HALF_A, D_HID), jnp.float32),
        pltpu.VMEM((HALF_A, CNT_W), jnp.float32),
        pltpu.VMEM((HALF_A, CNT_W), jnp.float32),
        pltpu.VMEM((HALF_A, D_HID), jnp.float32),
        pltpu.VMEM((D_HID,), jnp.float32),
        pltpu.VMEM_SHARED((ACC_ROWS, D_HID), jnp.float32),
        pltpu.VMEM_SHARED((ACC_ROWS, D_HID), jnp.float32),
    ],
    compiler_params=_SC_PARAMS,
)


def _proj_body(x_ref, wl_ref, wr_ref, p_ref, r_ref):
    x = x_ref[...]
    p_ref[...] = jnp.dot(x, wl_ref[...], preferred_element_type=jnp.float32)
    r_ref[...] = jnp.dot(x, wr_ref[...], preferred_element_type=jnp.float32)


_PROJ_BLOCK = 1264

_proj = pl.pallas_call(
    _proj_body,
    grid=(ACC_ROWS // _PROJ_BLOCK,),
    in_specs=[
        pl.BlockSpec((_PROJ_BLOCK, D_IN), lambda i: (i, 0)),
        pl.BlockSpec((D_IN, D_HID), lambda i: (0, 0)),
        pl.BlockSpec((D_IN, D_HID), lambda i: (0, 0)),
    ],
    out_specs=[
        pl.BlockSpec((_PROJ_BLOCK, D_HID), lambda i: (i, 0)),
        pl.BlockSpec((_PROJ_BLOCK, D_HID), lambda i: (i, 0)),
    ],
    out_shape=[
        jax.ShapeDtypeStruct((ACC_ROWS, D_HID), jnp.float32),
        jax.ShapeDtypeStruct((ACC_ROWS, D_HID), jnp.float32),
    ],
)


def _out_body(sums_ref, inv_ref, h_ref, wl_ref, b_ref, wr_ref, o_ref):
    agg = (sums_ref[0] + sums_ref[1]) * inv_ref[:, 0:1]
    o = (jnp.dot(agg, wl_ref[...], preferred_element_type=jnp.float32)
         + b_ref[...]
         + jnp.dot(h_ref[...], wr_ref[...], preferred_element_type=jnp.float32))
    m = jnp.max(o, axis=1, keepdims=True)
    e = jnp.exp(o - m)
    o_ref[...] = o - m - jnp.log(jnp.sum(e, axis=1, keepdims=True))


_out_final = pl.pallas_call(
    _out_body,
    grid=(N_NODES // ROW_BLOCK,),
    in_specs=[
        pl.BlockSpec((NC, ROW_BLOCK, D_HID), lambda i: (0, i, 0)),
        pl.BlockSpec((ROW_BLOCK, CNT_W), lambda i: (i, 0)),
        pl.BlockSpec((ROW_BLOCK, D_HID), lambda i: (i, 0)),
        pl.BlockSpec((D_HID, D_OUT), lambda i: (0, 0)),
        pl.BlockSpec((1, D_OUT), lambda i: (0, 0)),
        pl.BlockSpec((D_HID, D_OUT), lambda i: (0, 0)),
    ],
    out_specs=pl.BlockSpec((ROW_BLOCK, D_OUT), lambda i: (i, 0)),
    out_shape=jax.ShapeDtypeStruct((N_NODES, D_OUT), jnp.float32),
)


def kernel(x, edge_index, W1_l, b1, W1_r, W2_l, b2, W2_r):
    ei = edge_index.astype(jnp.int32).reshape(2, NW, CHUNKS_PER_WORKER, CH)

    zeros_s = jnp.zeros((ACC_ROWS, D_HID), jnp.float32)
    zeros_c = jnp.zeros((ACC_ROWS, CNT_W), jnp.float32)
    ones_b = jnp.ones((CH, CNT_W), jnp.float32)

    p1, r1 = _proj(x, W1_l, W1_r)
    sums1, cnt = _seg_sum_counts(p1, ei, zeros_s, zeros_c, ones_b)
    sums2, h, inv = _layer2(sums1, cnt, r1, b1, ei, zeros_s)
    return _out_final(sums2, inv, h, W2_l, b2.reshape(1, D_OUT), W2_r)


# speedup vs baseline: 1.1666x; 1.0405x over previous
import functools

import jax
import jax.numpy as jnp
from jax import lax
from jax.experimental import pallas as pl
from jax.experimental.pallas import tpu as pltpu
from jax.experimental.pallas import tpu_sc as plsc

N_NODES = 10000
N_EDGES = 320000
D_IN = 128
D_HID = 32
D_OUT = 128

NC = 2
NS = 16
NW = NC * NS
CH = 80
CHUNKS_PER_WORKER = N_EDGES // (NW * CH)
ACC_ROWS = 10112
ROWS_PER_SUB = ACC_ROWS // NS
HALF_A = 320
HALF_B = ROWS_PER_SUB - HALF_A
CNT_W = 16
LANES = 16

ROW_BLOCK = 2000

_SC_PARAMS = pltpu.CompilerParams(use_tc_tiling_on_sc=False)


def _l1_body(table, ei_hbm, zeros_s, zeros_c, ones_hbm,
             out_sums, out_cnt,
             src_v, dst_v, rows_v, sem_g, ones_v, sem_o, sem_s, acc, acc_cnt):
    c = lax.axis_index("c")
    s = lax.axis_index("s")
    wid = s * NC + c

    row0 = s * ROWS_PER_SUB
    pltpu.sync_copy(zeros_s.at[pl.ds(row0, ROWS_PER_SUB)],
                    acc.at[pl.ds(row0, ROWS_PER_SUB)])
    pltpu.sync_copy(zeros_c.at[pl.ds(row0, ROWS_PER_SUB)],
                    acc_cnt.at[pl.ds(row0, ROWS_PER_SUB)])
    pltpu.sync_copy(ones_hbm, ones_v)
    pltpu.sync_copy(ei_hbm.at[0, wid], src_v)
    pltpu.sync_copy(ei_hbm.at[1, wid], dst_v)
    plsc.subcore_barrier()

    for _w in range(5):
        pltpu.async_copy(table.at[src_v.at[_w]], rows_v.at[_w], sem_g)

    def body(i, carry):
        p = lax.rem(i, 10)

        @pl.when(i + 5 < CHUNKS_PER_WORKER)
        def _prefetch():
            pltpu.async_copy(table.at[src_v.at[i + 5]],
                             rows_v.at[lax.rem(i + 5, 10)], sem_g)

        @pl.when(i >= 4)
        def _drain_scatter():
            q = lax.rem(i - 4, 10)
            pltpu.make_async_copy(rows_v.at[q], acc.at[dst_v.at[i - 4]],
                                  sem_s).wait()

        pltpu.make_async_copy(table.at[src_v.at[i]], rows_v.at[p],
                              sem_g).wait()
        pltpu.async_copy(ones_v, acc_cnt.at[dst_v.at[i]], sem_o, add=True)
        pltpu.async_copy(rows_v.at[p], acc.at[dst_v.at[i]], sem_s, add=True)
        return carry

    lax.fori_loop(0, CHUNKS_PER_WORKER, body, 0)

    def drain_s(i, carry):
        pltpu.make_async_copy(rows_v.at[0], acc.at[dst_v.at[0]], sem_s).wait()
        return carry
    lax.fori_loop(0, 4, drain_s, 0)

    def drain(i, carry):
        pltpu.make_async_copy(ones_v, acc_cnt.at[dst_v.at[0]], sem_o).wait()
        return carry
    lax.fori_loop(0, CHUNKS_PER_WORKER, drain, 0)

    plsc.subcore_barrier()

    pltpu.sync_copy(acc.at[pl.ds(row0, ROWS_PER_SUB)],
                    out_sums.at[c, pl.ds(row0, ROWS_PER_SUB)])
    pltpu.sync_copy(acc_cnt.at[pl.ds(row0, ROWS_PER_SUB)],
                    out_cnt.at[c, pl.ds(row0, ROWS_PER_SUB)])


_seg_sum_counts = pl.kernel(
    _l1_body,
    out_type=[
        jax.ShapeDtypeStruct((NC, ACC_ROWS, D_HID), jnp.float32),
        jax.ShapeDtypeStruct((NC, ACC_ROWS, CNT_W), jnp.float32),
    ],
    mesh=plsc.VectorSubcoreMesh(core_axis_name="c", subcore_axis_name="s"),
    scratch_types=[
        pltpu.VMEM((CHUNKS_PER_WORKER, CH), jnp.int32),
        pltpu.VMEM((CHUNKS_PER_WORKER, CH), jnp.int32),
        pltpu.VMEM((10, CH, D_HID), jnp.float32),
        pltpu.SemaphoreType.DMA,
        pltpu.VMEM((CH, CNT_W), jnp.float32),
        pltpu.SemaphoreType.DMA,
        pltpu.SemaphoreType.DMA,
        pltpu.VMEM_SHARED((ACC_ROWS, D_HID), jnp.float32),
        pltpu.VMEM_SHARED((ACC_ROWS, CNT_W), jnp.float32),
    ],
    compiler_params=_SC_PARAMS,
)


def _l2_body(sums1, cnt1, r1_hbm, b1_hbm, ei_hbm, zeros_s,
             out_sums, h_out, inv_out,
             src_v, dst_v, rows_v, sem_g, sem_p, sem_s,
             s0_v, s1_v, c0_v, c1_v, r_v, b_v, h_tab, acc):
    c = lax.axis_index("c")
    s = lax.axis_index("s")
    wid = s * NC + c

    row0 = s * ROWS_PER_SUB
    pltpu.sync_copy(zeros_s.at[pl.ds(row0, ROWS_PER_SUB)],
                    acc.at[pl.ds(row0, ROWS_PER_SUB)])
    pltpu.sync_copy(ei_hbm.at[0, wid], src_v)
    pltpu.sync_copy(ei_hbm.at[1, wid], dst_v)
    pltpu.sync_copy(b1_hbm, b_v)

    for k, hn in ((0, HALF_A), (1, HALF_B)):
        r0k = row0 + k * HALF_A
        pltpu.async_copy(sums1.at[0, pl.ds(r0k, hn)], s0_v.at[pl.ds(0, hn)],
                         sem_p)
        pltpu.async_copy(sums1.at[1, pl.ds(r0k, hn)], s1_v.at[pl.ds(0, hn)],
                         sem_p)
        pltpu.async_copy(cnt1.at[0, pl.ds(r0k, hn)], c0_v.at[pl.ds(0, hn)],
                         sem_p)
        pltpu.async_copy(cnt1.at[1, pl.ds(r0k, hn)], c1_v.at[pl.ds(0, hn)],
                         sem_p)
        pltpu.async_copy(r1_hbm.at[pl.ds(r0k, hn)], r_v.at[pl.ds(0, hn)],
                         sem_p)
        pltpu.make_async_copy(sums1.at[0, pl.ds(r0k, hn)],
                              s0_v.at[pl.ds(0, hn)], sem_p).wait()
        pltpu.make_async_copy(sums1.at[0, pl.ds(r0k, hn)],
                              s0_v.at[pl.ds(0, hn)], sem_p).wait()
        pltpu.make_async_copy(cnt1.at[0, pl.ds(r0k, hn)],
                              c0_v.at[pl.ds(0, hn)], sem_p).wait()
        pltpu.make_async_copy(cnt1.at[0, pl.ds(r0k, hn)],
                              c0_v.at[pl.ds(0, hn)], sem_p).wait()
        pltpu.make_async_copy(r1_hbm.at[pl.ds(r0k, hn)],
                              r_v.at[pl.ds(0, hn)], sem_p).wait()

        def compute(i, carry):
            cv = c0_v[i, :] + c1_v[i, :]
            inv = 1.0 / jnp.maximum(cv, 1.0)
            c0_v[i, :] = inv
            for j in (0, LANES):
                val = ((s0_v[i, pl.ds(j, LANES)] + s1_v[i, pl.ds(j, LANES)])
                       * inv
                       + b_v[pl.ds(j, LANES)]
                       + r_v[i, pl.ds(j, LANES)])
                r_v[i, pl.ds(j, LANES)] = jnp.maximum(val, 0.0)
            return carry

        lax.fori_loop(0, hn, compute, 0)

        pltpu.sync_copy(r_v.at[pl.ds(0, hn)], h_tab.at[pl.ds(r0k, hn)])

        @pl.when(c == 0)
        def _publish():
            pltpu.sync_copy(r_v.at[pl.ds(0, hn)], h_out.at[pl.ds(r0k, hn)])
            pltpu.sync_copy(c0_v.at[pl.ds(0, hn)], inv_out.at[pl.ds(r0k, hn)])

    plsc.subcore_barrier()

    for _w in range(5):
        pltpu.async_copy(h_tab.at[src_v.at[_w]], rows_v.at[_w], sem_g)

    def body(i, carry):
        p = lax.rem(i, 10)

        @pl.when(i + 5 < CHUNKS_PER_WORKER)
        def _prefetch():
            pltpu.async_copy(h_tab.at[src_v.at[i + 5]],
                             rows_v.at[lax.rem(i + 5, 10)], sem_g)

        @pl.when(i >= 4)
        def _drain_scatter():
            q = lax.rem(i - 4, 10)
            pltpu.make_async_copy(rows_v.at[q], acc.at[dst_v.at[i - 4]],
                                  sem_s).wait()

        pltpu.make_async_copy(h_tab.at[src_v.at[i]], rows_v.at[p],
                              sem_g).wait()
        pltpu.async_copy(rows_v.at[p], acc.at[dst_v.at[i]], sem_s, add=True)
        return carry

    lax.fori_loop(0, CHUNKS_PER_WORKER, body, 0)

    def drain_s(i, carry):
        pltpu.make_async_copy(rows_v.at[0], acc.at[dst_v.at[0]], sem_s).wait()
        return carry
    lax.fori_loop(0, 4, drain_s, 0)
    plsc.subcore_barrier()

    pltpu.sync_copy(acc.at[pl.ds(row0, ROWS_PER_SUB)],
                    out_sums.at[c, pl.ds(row0, ROWS_PER_SUB)])


_layer2 = pl.kernel(
    _l2_body,
    out_type=[
        jax.ShapeDtypeStruct((NC, ACC_ROWS, D_HID), jnp.float32),
        jax.ShapeDtypeStruct((ACC_ROWS, D_HID), jnp.float32),
        jax.ShapeDtypeStruct((ACC_ROWS, CNT_W), jnp.float32),
    ],
    mesh=plsc.VectorSubcoreMesh(core_axis_name="c", subcore_axis_name="s"),
    scratch_types=[
        pltpu.VMEM((CHUNKS_PER_WORKER, CH), jnp.int32),
        pltpu.VMEM((CHUNKS_PER_WORKER, CH), jnp.int32),
        pltpu.VMEM((10, CH, D_HID), jnp.float32),
        pltpu.SemaphoreType.DMA,
        pltpu.SemaphoreType.DMA,
        pltpu.SemaphoreType.DMA,
        pltpu.VMEM((HALF_A, D_HID), jnp.float32),
        pltpu.VMEM((HALF_A, D_HID), jnp.float32),
        pltpu.VMEM((HALF_A, CNT_W), jnp.float32),
        pltpu.VMEM((HALF_A, CNT_W), jnp.float32),
        pltpu.VMEM((HALF_A, D_HID), jnp.float32),
        pltpu.VMEM((D_HID,), jnp.float32),
        pltpu.VMEM_SHARED((ACC_ROWS, D_HID), jnp.float32),
        pltpu.VMEM_SHARED((ACC_ROWS, D_HID), jnp.float32),
    ],
    compiler_params=_SC_PARAMS,
)


def _proj_body(x_ref, wl_ref, wr_ref, p_ref, r_ref):
    x = x_ref[...]
    p_ref[...] = jnp.dot(x, wl_ref[...], preferred_element_type=jnp.float32)
    r_ref[...] = jnp.dot(x, wr_ref[...], preferred_element_type=jnp.float32)


_PROJ_BLOCK = 1264

_proj = pl.pallas_call(
    _proj_body,
    grid=(ACC_ROWS // _PROJ_BLOCK,),
    in_specs=[
        pl.BlockSpec((_PROJ_BLOCK, D_IN), lambda i: (i, 0)),
        pl.BlockSpec((D_IN, D_HID), lambda i: (0, 0)),
        pl.BlockSpec((D_IN, D_HID), lambda i: (0, 0)),
    ],
    out_specs=[
        pl.BlockSpec((_PROJ_BLOCK, D_HID), lambda i: (i, 0)),
        pl.BlockSpec((_PROJ_BLOCK, D_HID), lambda i: (i, 0)),
    ],
    out_shape=[
        jax.ShapeDtypeStruct((ACC_ROWS, D_HID), jnp.float32),
        jax.ShapeDtypeStruct((ACC_ROWS, D_HID), jnp.float32),
    ],
)


def _out_body(sums_ref, inv_ref, h_ref, wl_ref, b_ref, wr_ref, o_ref):
    agg = (sums_ref[0] + sums_ref[1]) * inv_ref[:, 0:1]
    o = (jnp.dot(agg, wl_ref[...], preferred_element_type=jnp.float32)
         + b_ref[...]
         + jnp.dot(h_ref[...], wr_ref[...], preferred_element_type=jnp.float32))
    m = jnp.max(o, axis=1, keepdims=True)
    e = jnp.exp(o - m)
    o_ref[...] = o - m - jnp.log(jnp.sum(e, axis=1, keepdims=True))


_out_final = pl.pallas_call(
    _out_body,
    grid=(N_NODES // ROW_BLOCK,),
    in_specs=[
        pl.BlockSpec((NC, ROW_BLOCK, D_HID), lambda i: (0, i, 0)),
        pl.BlockSpec((ROW_BLOCK, CNT_W), lambda i: (i, 0)),
        pl.BlockSpec((ROW_BLOCK, D_HID), lambda i: (i, 0)),
        pl.BlockSpec((D_HID, D_OUT), lambda i: (0, 0)),
        pl.BlockSpec((1, D_OUT), lambda i: (0, 0)),
        pl.BlockSpec((D_HID, D_OUT), lambda i: (0, 0)),
    ],
    out_specs=pl.BlockSpec((ROW_BLOCK, D_OUT), lambda i: (i, 0)),
    out_shape=jax.ShapeDtypeStruct((N_NODES, D_OUT), jnp.float32),
)


def kernel(x, edge_index, W1_l, b1, W1_r, W2_l, b2, W2_r):
    ei = edge_index.astype(jnp.int32).reshape(2, NW, CHUNKS_PER_WORKER, CH)

    zeros_s = jnp.zeros((ACC_ROWS, D_HID), jnp.float32)
    zeros_c = jnp.zeros((ACC_ROWS, CNT_W), jnp.float32)
    ones_b = jnp.ones((CH, CNT_W), jnp.float32)

    p1, r1 = _proj(x, W1_l, W1_r)
    sums1, cnt = _seg_sum_counts(p1, ei, zeros_s, zeros_c, ones_b)
    sums2, h, inv = _layer2(sums1, cnt, r1, b1, ei, zeros_s)
    return _out_final(sums2, inv, h, W2_l, b2.reshape(1, D_OUT), W2_r)
